# baseline (speedup 1.0000x reference)
import functools

import jax
import jax.numpy as jnp
from jax import lax
from jax.experimental import pallas as pl
from jax.experimental.pallas import tpu as pltpu
from jax.experimental.pallas import tpu_sc as plsc

N = 10000
NP = 10240
E = 160000
NC = 2
NS = 16
NW = NC * NS
EPT = E // NW
KP = 128
NCHUNK = 40
PAD = NCHUNK * KP - EPT
RPT = NP // NS

_MESH = plsc.VectorSubcoreMesh(core_axis_name="c", subcore_axis_name="s")


def _make_deg_kernel():
  @functools.partial(
      pl.kernel,
      out_type=jax.ShapeDtypeStruct((NC, NP), jnp.float32),
      mesh=_MESH,
      compiler_params=pltpu.CompilerParams(use_tc_tiling_on_sc=False),
      scratch_types=[
          pltpu.VMEM((NCHUNK, KP), jnp.int32),
          pltpu.VMEM((KP,), jnp.float32),
          pltpu.VMEM_SHARED((NP,), jnp.float32),
      ],
  )
  def deg_kernel(col_hbm, ones_hbm, zeros_hbm, out_hbm, col_v, ones_v, dacc):
    c = lax.axis_index("c")
    s = lax.axis_index("s")
    t = s * NC + c
    base = s * RPT
    pltpu.sync_copy(zeros_hbm.at[pl.ds(base, RPT)], dacc.at[pl.ds(base, RPT)])
    pltpu.sync_copy(col_hbm.at[t], col_v)
    pltpu.sync_copy(ones_hbm, ones_v)
    plsc.subcore_barrier()

    def body(j, carry):
      pltpu.sync_copy(ones_v, dacc.at[col_v.at[j]], add=True)
      return carry

    lax.fori_loop(0, NCHUNK, body, 0)
    plsc.subcore_barrier()
    pltpu.sync_copy(dacc.at[pl.ds(base, RPT)],
                    out_hbm.at[c].at[pl.ds(base, RPT)])

  return deg_kernel


def _make_agg_kernel(D, tiled):
  @functools.partial(
      pl.kernel,
      out_type=jax.ShapeDtypeStruct((NC, NP, D), jnp.float32),
      mesh=_MESH,
      compiler_params=pltpu.CompilerParams(use_tc_tiling_on_sc=tiled),
      scratch_types=[
          pltpu.VMEM((NCHUNK, KP), jnp.int32),
          pltpu.VMEM((NCHUNK, KP), jnp.int32),
          pltpu.VMEM((2 * KP, D), jnp.float32),
          pltpu.VMEM_SHARED((NP, D), jnp.float32),
          pltpu.SemaphoreType.DMA,
          pltpu.SemaphoreType.DMA,
      ],
  )
  def agg_kernel(row_hbm, col_hbm, src_hbm, zeros_hbm, out_hbm,
                 row_v, col_v, buf, acc, gsem, ssem):
    c = lax.axis_index("c")
    s = lax.axis_index("s")
    t = s * NC + c
    base = s * RPT

    @pl.when(c == 0)
    def _():
      pltpu.sync_copy(src_hbm.at[pl.ds(base, RPT)], acc.at[pl.ds(base, RPT)])

    @pl.when(c != 0)
    def _():
      pltpu.sync_copy(zeros_hbm.at[pl.ds(base, RPT)], acc.at[pl.ds(base, RPT)])

    pltpu.sync_copy(row_hbm.at[t], row_v)
    pltpu.sync_copy(col_hbm.at[t], col_v)
    plsc.subcore_barrier()

    def gissue(r, hoff):
      pltpu.async_copy(src_hbm.at[row_v.at[r]],
                       buf.at[pl.ds(hoff, KP)], gsem)

    def gwait(r, hoff):
      pltpu.make_async_copy(src_hbm.at[row_v.at[r]],
                            buf.at[pl.ds(hoff, KP)], gsem).wait()

    def sissue(r, hoff):
      pltpu.async_copy(buf.at[pl.ds(hoff, KP)],
                       acc.at[col_v.at[r]], ssem, add=True)

    def swait(r, hoff):
      pltpu.make_async_copy(buf.at[pl.ds(hoff, KP)],
                            acc.at[col_v.at[r]], ssem).wait()

    gissue(0, 0)

    def round_body(r, carry):
      hoff = (r % 2) * KP
      ooff = KP - hoff
      gwait(r, hoff)

      @pl.when(r >= 1)
      def _():
        swait(r - 1, ooff)

      @pl.when(r + 1 < NCHUNK)
      def _():
        gissue(r + 1, ooff)

      sissue(r, hoff)
      return carry

    lax.fori_loop(0, NCHUNK, round_body, 0)
    swait(NCHUNK - 1, ((NCHUNK - 1) % 2) * KP)
    plsc.subcore_barrier()
    pltpu.sync_copy(acc.at[pl.ds(base, RPT)],
                    out_hbm.at[c].at[pl.ds(base, RPT)])

  return agg_kernel


_deg_kernel = _make_deg_kernel()
_agg128 = _make_agg_kernel(128, False)
_agg48 = _make_agg_kernel(48, False)


_RBLK = 640
_NBLK = NP // _RBLK


def _prescale_body(deg_ref, x_ref, xs_ref, dinv_ref):
  deg = deg_ref[:, 0:1] + deg_ref[:, 1:2] + 1.0
  dinv = lax.rsqrt(deg)
  dinv_ref[...] = dinv
  xs_ref[...] = x_ref[...] * dinv


def _tc_prescale(deg_t, x_pad):
  return pl.pallas_call(
      _prescale_body,
      grid=(_NBLK,),
      in_specs=[
          pl.BlockSpec((_RBLK, NC), lambda i: (i, 0)),
          pl.BlockSpec((_RBLK, 128), lambda i: (i, 0)),
      ],
      out_specs=[
          pl.BlockSpec((_RBLK, 128), lambda i: (i, 0)),
          pl.BlockSpec((_RBLK, 1), lambda i: (i, 0)),
      ],
      out_shape=[
          jax.ShapeDtypeStruct((NP, 128), jnp.float32),
          jax.ShapeDtypeStruct((NP, 1), jnp.float32),
      ],
  )(deg_t, x_pad)


def _mm_body(p_ref, dinv_ref, w1_ref, b1_ref, w2_ref, ys_ref):
  dinv = dinv_ref[...]
  a = (p_ref[0] + p_ref[1]) * dinv
  h = jnp.dot(a, w1_ref[...], preferred_element_type=jnp.float32)
  h = jnp.maximum(h + b1_ref[...], 0.0)
  y = jnp.dot(h, w2_ref[...], preferred_element_type=jnp.float32)
  ys_ref[...] = y * dinv


def _tc_mm(p, dinv, w1, b1, w2p):
  return pl.pallas_call(
      _mm_body,
      grid=(_NBLK,),
      in_specs=[
          pl.BlockSpec((NC, _RBLK, 128), lambda i: (0, i, 0)),
          pl.BlockSpec((_RBLK, 1), lambda i: (i, 0)),
          pl.BlockSpec((128, 2048), lambda i: (0, 0)),
          pl.BlockSpec((1, 2048), lambda i: (0, 0)),
          pl.BlockSpec((2048, 48), lambda i: (0, 0)),
      ],
      out_specs=pl.BlockSpec((_RBLK, 48), lambda i: (i, 0)),
      out_shape=jax.ShapeDtypeStruct((NP, 48), jnp.float32),
  )(p, dinv, w1, b1, w2p)


def _final_body(q_ref, dinv_ref, b2_ref, out_ref):
  out_ref[...] = (q_ref[0] + q_ref[1]) * dinv_ref[...] + b2_ref[...]


def _tc_final(q, dinv, b2p):
  return pl.pallas_call(
      _final_body,
      grid=(_NBLK,),
      in_specs=[
          pl.BlockSpec((NC, _RBLK, 48), lambda i: (0, i, 0)),
          pl.BlockSpec((_RBLK, 1), lambda i: (i, 0)),
          pl.BlockSpec((1, 48), lambda i: (0, 0)),
      ],
      out_specs=pl.BlockSpec((_RBLK, 48), lambda i: (i, 0)),
      out_shape=jax.ShapeDtypeStruct((NP, 48), jnp.float32),
  )(q, dinv, b2p)


def kernel(x, edge_index, W1, b1, W2, b2):
  ei = edge_index.astype(jnp.int32)
  rowp = jnp.concatenate(
      [ei[0].reshape(NW, EPT), jnp.zeros((NW, PAD), jnp.int32)],
      axis=1).reshape(NW, NCHUNK, KP)
  colp = jnp.concatenate(
      [ei[1].reshape(NW, EPT), jnp.full((NW, PAD), NP - 1, jnp.int32)],
      axis=1).reshape(NW, NCHUNK, KP)
  x_pad = jnp.pad(x, ((0, NP - N), (0, 0)))
  w2p = jnp.pad(W2, ((0, 0), (0, 48 - W2.shape[1])))
  b1r = b1.reshape(1, 2048)
  b2p = jnp.pad(b2, (0, 48 - b2.shape[0])).reshape(1, 48)
  ones_k = jnp.ones((KP,), jnp.float32)
  z1 = jnp.zeros((NP,), jnp.float32)
  z128 = jnp.zeros((NP, 128), jnp.float32)
  z48 = jnp.zeros((NP, 48), jnp.float32)

  degp = _deg_kernel(colp, ones_k, z1)
  deg_t = degp.T
  xs, dinv = _tc_prescale(deg_t, x_pad)
  p = _agg128(rowp, colp, xs, z128)
  ys = _tc_mm(p, dinv, W1, b1r, w2p)
  q = _agg48(rowp, colp, ys, z48)
  outp = _tc_final(q, dinv, b2p)
  return outp[:N, :40]


# speedup vs baseline: 1.6597x; 1.6597x over previous
import functools

import jax
import jax.numpy as jnp
from jax import lax
from jax.experimental import pallas as pl
from jax.experimental.pallas import tpu as pltpu
from jax.experimental.pallas import tpu_sc as plsc

N = 10000
NP = 10240
E = 160000
NC = 2
NS = 16
NW = NC * NS
EPT = E // NW
K = 40
NCHUNK = EPT // K
RB = 5
NROUND = NCHUNK // RB
NB = 4
RPT = NP // NS

_MESH = plsc.VectorSubcoreMesh(core_axis_name="c", subcore_axis_name="s")


def _make_deg_kernel():
  DDEEP = 4

  @functools.partial(
      pl.kernel,
      out_type=jax.ShapeDtypeStruct((NC, NP), jnp.float32),
      mesh=_MESH,
      compiler_params=pltpu.CompilerParams(use_tc_tiling_on_sc=False),
      scratch_types=[
          pltpu.VMEM((NCHUNK, K), jnp.int32),
          pltpu.VMEM((K,), jnp.float32),
          pltpu.VMEM_SHARED((NP,), jnp.float32),
          pltpu.SemaphoreType.DMA,
      ],
  )
  def deg_kernel(col_hbm, ones_hbm, zeros_hbm, out_hbm, col_v, ones_v, dacc,
                 sem):
    c = lax.axis_index("c")
    s = lax.axis_index("s")
    t = s * NC + c
    base = s * RPT
    pltpu.sync_copy(zeros_hbm.at[pl.ds(base, RPT)], dacc.at[pl.ds(base, RPT)])
    pltpu.sync_copy(col_hbm.at[t], col_v)
    pltpu.sync_copy(ones_hbm, ones_v)
    plsc.subcore_barrier()

    for j in range(DDEEP):
      pltpu.async_copy(ones_v, dacc.at[col_v.at[j]], sem, add=True)

    def body(j, carry):
      @pl.when(j + DDEEP < NCHUNK)
      def _():
        pltpu.async_copy(ones_v, dacc.at[col_v.at[j + DDEEP]], sem, add=True)

      pltpu.make_async_copy(ones_v, dacc.at[col_v.at[j]], sem).wait()
      return carry

    lax.fori_loop(0, NCHUNK, body, 0)
    plsc.subcore_barrier()
    pltpu.sync_copy(dacc.at[pl.ds(base, RPT)],
                    out_hbm.at[c].at[pl.ds(base, RPT)])

  return deg_kernel


def _make_agg_kernel(D, nphase):
  CH = RB * K

  @functools.partial(
      pl.kernel,
      out_type=jax.ShapeDtypeStruct((nphase, NC, NP, D), jnp.float32),
      mesh=_MESH,
      compiler_params=pltpu.CompilerParams(use_tc_tiling_on_sc=False),
      scratch_types=[
          pltpu.VMEM((NCHUNK, K), jnp.int32),
          pltpu.VMEM((NCHUNK, K), jnp.int32),
          pltpu.VMEM((NB * CH, D), jnp.float32),
          pltpu.VMEM_SHARED((NP, D), jnp.float32),
          pltpu.SemaphoreType.DMA,
          pltpu.SemaphoreType.DMA,
      ],
  )
  def agg_kernel(row_hbm, col_hbm, src_hbm, zeros_hbm, out_hbm,
                 row_v, col_v, buf, acc, gsem, ssem):
    c = lax.axis_index("c")
    s = lax.axis_index("s")
    t = s * NC + c
    base = s * RPT

    pltpu.sync_copy(row_hbm.at[t], row_v)
    pltpu.sync_copy(col_hbm.at[t], col_v)

    for p in range(nphase):
      @pl.when(c == 0)
      def _():
        pltpu.sync_copy(src_hbm.at[p].at[pl.ds(base, RPT)],
                        acc.at[pl.ds(base, RPT)])

      @pl.when(c != 0)
      def _():
        pltpu.sync_copy(zeros_hbm.at[pl.ds(base, RPT)],
                        acc.at[pl.ds(base, RPT)])

      plsc.subcore_barrier()

      def gissue(r, hoff):
        for b in range(RB):
          pltpu.async_copy(src_hbm.at[p].at[row_v.at[r * RB + b]],
                           buf.at[pl.ds(hoff + b * K, K)], gsem)

      def gwait(r, hoff):
        for b in range(RB):
          pltpu.make_async_copy(src_hbm.at[p].at[row_v.at[r * RB + b]],
                                buf.at[pl.ds(hoff + b * K, K)], gsem).wait()

      def sissue(r, hoff):
        for b in range(RB):
          pltpu.async_copy(buf.at[pl.ds(hoff + b * K, K)],
                           acc.at[col_v.at[r * RB + b]], ssem, add=True)

      def swait(r, hoff):
        for b in range(RB):
          pltpu.make_async_copy(buf.at[pl.ds(hoff + b * K, K)],
                                acc.at[col_v.at[r * RB + b]], ssem).wait()

      gissue(0, 0)

      def round_body(r, carry):
        hoff = (r % NB) * CH

        gwait(r, hoff)

        @pl.when(r + 1 >= NB)
        def _():
          swait(r + 1 - NB, ((r + 1 - NB) % NB) * CH)

        @pl.when(r + 1 < NROUND)
        def _():
          gissue(r + 1, ((r + 1) % NB) * CH)

        sissue(r, hoff)
        return carry

      lax.fori_loop(0, NROUND, round_body, 0)
      for j in range(max(0, NROUND - NB + 1), NROUND):
        swait(j, (j % NB) * CH)
      plsc.subcore_barrier()
      pltpu.sync_copy(acc.at[pl.ds(base, RPT)],
                      out_hbm.at[p].at[c].at[pl.ds(base, RPT)])

  return agg_kernel


_deg_kernel = _make_deg_kernel()
_agg64x2 = _make_agg_kernel(64, 2)
_agg48 = _make_agg_kernel(48, 1)


_RBLK = 640
_NBLK = NP // _RBLK


def _prescale_body(deg_ref, x_ref, xs_ref, dinv_ref):
  deg = deg_ref[:, 0:1] + deg_ref[:, 1:2] + 1.0
  dinv = lax.rsqrt(deg)
  dinv_ref[...] = dinv
  xs_ref[0] = x_ref[:, :64] * dinv
  xs_ref[1] = x_ref[:, 64:] * dinv


def _tc_prescale(deg_t, x_pad):
  return pl.pallas_call(
      _prescale_body,
      grid=(_NBLK,),
      in_specs=[
          pl.BlockSpec((_RBLK, NC), lambda i: (i, 0)),
          pl.BlockSpec((_RBLK, 128), lambda i: (i, 0)),
      ],
      out_specs=[
          pl.BlockSpec((2, _RBLK, 64), lambda i: (0, i, 0)),
          pl.BlockSpec((_RBLK, 1), lambda i: (i, 0)),
      ],
      out_shape=[
          jax.ShapeDtypeStruct((2, NP, 64), jnp.float32),
          jax.ShapeDtypeStruct((NP, 1), jnp.float32),
      ],
  )(deg_t, x_pad)


def _mm_body(p_ref, dinv_ref, w1_ref, b1_ref, w2_ref, ys_ref):
  dinv = dinv_ref[...]
  a = jnp.concatenate([p_ref[0, 0] + p_ref[0, 1],
                       p_ref[1, 0] + p_ref[1, 1]], axis=1) * dinv
  h = jnp.dot(a, w1_ref[...], preferred_element_type=jnp.float32)
  h = jnp.maximum(h + b1_ref[...], 0.0)
  y = jnp.dot(h, w2_ref[...], preferred_element_type=jnp.float32)
  ys_ref[...] = y * dinv


def _tc_mm(p, dinv, w1, b1, w2p):
  return pl.pallas_call(
      _mm_body,
      grid=(_NBLK,),
      in_specs=[
          pl.BlockSpec((2, NC, _RBLK, 64), lambda i: (0, 0, i, 0)),
          pl.BlockSpec((_RBLK, 1), lambda i: (i, 0)),
          pl.BlockSpec((128, 2048), lambda i: (0, 0)),
          pl.BlockSpec((1, 2048), lambda i: (0, 0)),
          pl.BlockSpec((2048, 48), lambda i: (0, 0)),
      ],
      out_specs=pl.BlockSpec((_RBLK, 48), lambda i: (i, 0)),
      out_shape=jax.ShapeDtypeStruct((NP, 48), jnp.float32),
  )(p, dinv, w1, b1, w2p)


def _final_body(q_ref, dinv_ref, b2_ref, out_ref):
  out_ref[...] = (q_ref[0] + q_ref[1]) * dinv_ref[...] + b2_ref[...]


def _tc_final(q, dinv, b2p):
  return pl.pallas_call(
      _final_body,
      grid=(_NBLK,),
      in_specs=[
          pl.BlockSpec((NC, _RBLK, 48), lambda i: (0, i, 0)),
          pl.BlockSpec((_RBLK, 1), lambda i: (i, 0)),
          pl.BlockSpec((1, 48), lambda i: (0, 0)),
      ],
      out_specs=pl.BlockSpec((_RBLK, 48), lambda i: (i, 0)),
      out_shape=jax.ShapeDtypeStruct((NP, 48), jnp.float32),
  )(q, dinv, b2p)


def kernel(x, edge_index, W1, b1, W2, b2):
  ei = edge_index.astype(jnp.int32)
  row2 = ei[0].reshape(NW, NCHUNK, K)
  col2 = ei[1].reshape(NW, NCHUNK, K)
  x_pad = jnp.pad(x, ((0, NP - N), (0, 0)))
  w2p = jnp.pad(W2, ((0, 0), (0, 48 - W2.shape[1])))
  b1r = b1.reshape(1, 2048)
  b2p = jnp.pad(b2, (0, 48 - b2.shape[0])).reshape(1, 48)
  ones_k = jnp.ones((K,), jnp.float32)
  z1 = jnp.zeros((NP,), jnp.float32)
  z64 = jnp.zeros((NP, 64), jnp.float32)
  z48 = jnp.zeros((NP, 48), jnp.float32)

  degp = _deg_kernel(col2, ones_k, z1)
  deg_t = degp.T
  xs2, dinv = _tc_prescale(deg_t, x_pad)
  p = _agg64x2(row2, col2, xs2, z64)
  ys = _tc_mm(p, dinv, W1, b1r, w2p)
  q = _agg48(row2, col2, ys[None], z48)
  outp = _tc_final(q[0], dinv, b2p)
  return outp[:N, :40]


# speedup vs baseline: 1.6616x; 1.0011x over previous
import functools

import jax
import jax.numpy as jnp
from jax import lax
from jax.experimental import pallas as pl
from jax.experimental.pallas import tpu as pltpu
from jax.experimental.pallas import tpu_sc as plsc

N = 10000
NP = 10240
E = 160000
NC = 2
NS = 16
NW = NC * NS
EPT = E // NW
K = 40
NCHUNK = EPT // K
RB = 5
NROUND = NCHUNK // RB
NB = 4
RPT = NP // NS

_MESH = plsc.VectorSubcoreMesh(core_axis_name="c", subcore_axis_name="s")


def _make_deg_kernel():
  DDEEP = 4

  @functools.partial(
      pl.kernel,
      out_type=jax.ShapeDtypeStruct((NC, NP), jnp.float32),
      mesh=_MESH,
      compiler_params=pltpu.CompilerParams(use_tc_tiling_on_sc=False),
      scratch_types=[
          pltpu.VMEM((NCHUNK, K), jnp.int32),
          pltpu.VMEM((K,), jnp.float32),
          pltpu.VMEM_SHARED((NP,), jnp.float32),
          pltpu.SemaphoreType.DMA,
      ],
  )
  def deg_kernel(col_hbm, ones_hbm, zeros_hbm, out_hbm, col_v, ones_v, dacc,
                 sem):
    c = lax.axis_index("c")
    s = lax.axis_index("s")
    t = s * NC + c
    base = s * RPT
    pltpu.sync_copy(zeros_hbm.at[pl.ds(base, RPT)], dacc.at[pl.ds(base, RPT)])
    pltpu.sync_copy(col_hbm.at[t], col_v)
    pltpu.sync_copy(ones_hbm, ones_v)
    plsc.subcore_barrier()

    for j in range(DDEEP):
      pltpu.async_copy(ones_v, dacc.at[col_v.at[j]], sem, add=True)

    def body(j, carry):
      @pl.when(j + DDEEP < NCHUNK)
      def _():
        pltpu.async_copy(ones_v, dacc.at[col_v.at[j + DDEEP]], sem, add=True)

      pltpu.make_async_copy(ones_v, dacc.at[col_v.at[j]], sem).wait()
      return carry

    lax.fori_loop(0, NCHUNK, body, 0)
    plsc.subcore_barrier()
    pltpu.sync_copy(dacc.at[pl.ds(base, RPT)],
                    out_hbm.at[c].at[pl.ds(base, RPT)])

  return deg_kernel


def _make_agg_kernel(D, nphase):
  CH = RB * K

  @functools.partial(
      pl.kernel,
      out_type=jax.ShapeDtypeStruct((nphase, NC, NP, D), jnp.float32),
      mesh=_MESH,
      compiler_params=pltpu.CompilerParams(use_tc_tiling_on_sc=False),
      scratch_types=[
          pltpu.VMEM((NCHUNK, K), jnp.int32),
          pltpu.VMEM((NCHUNK, K), jnp.int32),
          pltpu.VMEM((NB * CH, D), jnp.float32),
          pltpu.VMEM_SHARED((NP, D), jnp.float32),
          pltpu.SemaphoreType.DMA,
          pltpu.SemaphoreType.DMA,
      ],
  )
  def agg_kernel(row_hbm, col_hbm, src_hbm, zeros_hbm, out_hbm,
                 row_v, col_v, buf, acc, gsem, ssem):
    c = lax.axis_index("c")
    s = lax.axis_index("s")
    t = s * NC + c
    base = s * RPT

    pltpu.sync_copy(row_hbm.at[t], row_v)
    pltpu.sync_copy(col_hbm.at[t], col_v)

    for p in range(nphase):
      @pl.when(c == 0)
      def _():
        pltpu.sync_copy(src_hbm.at[p].at[pl.ds(base, RPT)],
                        acc.at[pl.ds(base, RPT)])

      @pl.when(c != 0)
      def _():
        pltpu.sync_copy(zeros_hbm.at[pl.ds(base, RPT)],
                        acc.at[pl.ds(base, RPT)])

      plsc.subcore_barrier()

      def gissue(r, hoff):
        for b in range(RB):
          pltpu.async_copy(src_hbm.at[p].at[row_v.at[r * RB + b]],
                           buf.at[pl.ds(hoff + b * K, K)], gsem)

      def gwait(r, hoff):
        for b in range(RB):
          pltpu.make_async_copy(src_hbm.at[p].at[row_v.at[r * RB + b]],
                                buf.at[pl.ds(hoff + b * K, K)], gsem).wait()

      def sissue(r, hoff):
        for b in range(RB):
          pltpu.async_copy(buf.at[pl.ds(hoff + b * K, K)],
                           acc.at[col_v.at[r * RB + b]], ssem, add=True)

      def swait(r, hoff):
        for b in range(RB):
          pltpu.make_async_copy(buf.at[pl.ds(hoff + b * K, K)],
                                acc.at[col_v.at[r * RB + b]], ssem).wait()

      gissue(0, 0)

      def round_body(r, carry):
        hoff = (r % NB) * CH

        gwait(r, hoff)

        @pl.when(r + 1 >= NB)
        def _():
          swait(r + 1 - NB, ((r + 1 - NB) % NB) * CH)

        @pl.when(r + 1 < NROUND)
        def _():
          gissue(r + 1, ((r + 1) % NB) * CH)

        sissue(r, hoff)
        return carry

      lax.fori_loop(0, NROUND, round_body, 0)
      for j in range(max(0, NROUND - NB + 1), NROUND):
        swait(j, (j % NB) * CH)
      plsc.subcore_barrier()
      pltpu.sync_copy(acc.at[pl.ds(base, RPT)],
                      out_hbm.at[p].at[c].at[pl.ds(base, RPT)])

  return agg_kernel


_deg_kernel = _make_deg_kernel()
_agg64x2 = _make_agg_kernel(64, 2)
_agg40 = _make_agg_kernel(40, 1)


_RBLK = 640
_NBLK = NP // _RBLK


def _prescale_body(deg_ref, x_ref, xs_ref, dinv_ref):
  deg = deg_ref[:, 0:1] + deg_ref[:, 1:2] + 1.0
  dinv = lax.rsqrt(deg)
  dinv_ref[...] = dinv
  xs_ref[0] = x_ref[:, :64] * dinv
  xs_ref[1] = x_ref[:, 64:] * dinv


def _tc_prescale(deg_t, x_pad):
  return pl.pallas_call(
      _prescale_body,
      grid=(_NBLK,),
      in_specs=[
          pl.BlockSpec((_RBLK, NC), lambda i: (i, 0)),
          pl.BlockSpec((_RBLK, 128), lambda i: (i, 0)),
      ],
      out_specs=[
          pl.BlockSpec((2, _RBLK, 64), lambda i: (0, i, 0)),
          pl.BlockSpec((_RBLK, 1), lambda i: (i, 0)),
      ],
      out_shape=[
          jax.ShapeDtypeStruct((2, NP, 64), jnp.float32),
          jax.ShapeDtypeStruct((NP, 1), jnp.float32),
      ],
  )(deg_t, x_pad)


def _mm_body(p_ref, dinv_ref, w1_ref, b1_ref, w2_ref, ys_ref):
  dinv = dinv_ref[...]
  a = jnp.concatenate([p_ref[0, 0] + p_ref[0, 1],
                       p_ref[1, 0] + p_ref[1, 1]], axis=1) * dinv
  h = jnp.dot(a.astype(jnp.bfloat16), w1_ref[...].astype(jnp.bfloat16),
              preferred_element_type=jnp.float32)
  h = jnp.maximum(h + b1_ref[...], 0.0)
  y = jnp.dot(h.astype(jnp.bfloat16), w2_ref[...].astype(jnp.bfloat16),
              preferred_element_type=jnp.float32)
  ys_ref[...] = y * dinv


def _tc_mm(p, dinv, w1, b1, w2):
  return pl.pallas_call(
      _mm_body,
      grid=(_NBLK,),
      in_specs=[
          pl.BlockSpec((2, NC, _RBLK, 64), lambda i: (0, 0, i, 0)),
          pl.BlockSpec((_RBLK, 1), lambda i: (i, 0)),
          pl.BlockSpec((128, 2048), lambda i: (0, 0)),
          pl.BlockSpec((1, 2048), lambda i: (0, 0)),
          pl.BlockSpec((2048, 40), lambda i: (0, 0)),
      ],
      out_specs=pl.BlockSpec((_RBLK, 40), lambda i: (i, 0)),
      out_shape=jax.ShapeDtypeStruct((NP, 40), jnp.float32),
  )(p, dinv, w1, b1, w2)


def _final_body(q_ref, dinv_ref, b2_ref, out_ref):
  out_ref[...] = (q_ref[0] + q_ref[1]) * dinv_ref[...] + b2_ref[...]


def _tc_final(q, dinv, b2p):
  return pl.pallas_call(
      _final_body,
      grid=(_NBLK,),
      in_specs=[
          pl.BlockSpec((NC, _RBLK, 40), lambda i: (0, i, 0)),
          pl.BlockSpec((_RBLK, 1), lambda i: (i, 0)),
          pl.BlockSpec((1, 40), lambda i: (0, 0)),
      ],
      out_specs=pl.BlockSpec((_RBLK, 40), lambda i: (i, 0)),
      out_shape=jax.ShapeDtypeStruct((NP, 40), jnp.float32),
  )(q, dinv, b2p)


def kernel(x, edge_index, W1, b1, W2, b2):
  ei = edge_index.astype(jnp.int32)
  row2 = ei[0].reshape(NW, NCHUNK, K)
  col2 = ei[1].reshape(NW, NCHUNK, K)
  x_pad = jnp.pad(x, ((0, NP - N), (0, 0)))
  b1r = b1.reshape(1, 2048)
  b2r = b2.reshape(1, 40)
  ones_k = jnp.ones((K,), jnp.float32)
  z1 = jnp.zeros((NP,), jnp.float32)
  z64 = jnp.zeros((NP, 64), jnp.float32)
  z40 = jnp.zeros((NP, 40), jnp.float32)

  degp = _deg_kernel(col2, ones_k, z1)
  deg_t = degp.T
  xs2, dinv = _tc_prescale(deg_t, x_pad)
  p = _agg64x2(row2, col2, xs2, z64)
  ys = _tc_mm(p, dinv, W1, b1r, W2)
  q = _agg40(row2, col2, ys[None], z40)
  outp = _tc_final(q[0], dinv, b2r)
  return outp[:N]


# speedup vs baseline: 1.8972x; 1.1418x over previous
import functools

import jax
import jax.numpy as jnp
from jax import lax
from jax.experimental import pallas as pl
from jax.experimental.pallas import tpu as pltpu
from jax.experimental.pallas import tpu_sc as plsc

N = 10000
NP = 10240
E = 160000
NC = 2
NS = 16
NW = NC * NS
EPT = E // NW
K = 40
NCHUNK = EPT // K
RB = 5
NROUND = NCHUNK // RB
NB = 4
RPT = NP // NS

_MESH = plsc.VectorSubcoreMesh(core_axis_name="c", subcore_axis_name="s")


def _make_deg_kernel():
  DDEEP = 4

  @functools.partial(
      pl.kernel,
      out_type=jax.ShapeDtypeStruct((NC, NP), jnp.float32),
      mesh=_MESH,
      compiler_params=pltpu.CompilerParams(use_tc_tiling_on_sc=False),
      scratch_types=[
          pltpu.VMEM((NCHUNK, K), jnp.int32),
          pltpu.VMEM((K,), jnp.float32),
          pltpu.VMEM_SHARED((NP,), jnp.float32),
          pltpu.SemaphoreType.DMA,
      ],
  )
  def deg_kernel(col_hbm, ones_hbm, zeros_hbm, out_hbm, col_v, ones_v, dacc,
                 sem):
    c = lax.axis_index("c")
    s = lax.axis_index("s")
    t = s * NC + c
    base = s * RPT
    pltpu.sync_copy(zeros_hbm.at[pl.ds(base, RPT)], dacc.at[pl.ds(base, RPT)])
    pltpu.sync_copy(col_hbm.at[t], col_v)
    pltpu.sync_copy(ones_hbm, ones_v)
    plsc.subcore_barrier()

    for j in range(DDEEP):
      pltpu.async_copy(ones_v, dacc.at[col_v.at[j]], sem, add=True)

    def body(j, carry):
      @pl.when(j + DDEEP < NCHUNK)
      def _():
        pltpu.async_copy(ones_v, dacc.at[col_v.at[j + DDEEP]], sem, add=True)

      pltpu.make_async_copy(ones_v, dacc.at[col_v.at[j]], sem).wait()
      return carry

    lax.fori_loop(0, NCHUNK, body, 0)
    plsc.subcore_barrier()
    pltpu.sync_copy(dacc.at[pl.ds(base, RPT)],
                    out_hbm.at[c].at[pl.ds(base, RPT)])

  return deg_kernel


def _make_agg_kernel(D):
  GD = 3
  SD = 3
  NBUF = GD + SD

  @functools.partial(
      pl.kernel,
      out_type=jax.ShapeDtypeStruct((NC, NP, D), jnp.float32),
      mesh=_MESH,
      compiler_params=pltpu.CompilerParams(use_tc_tiling_on_sc=False),
      scratch_types=[
          pltpu.VMEM((NCHUNK, K), jnp.int32),
          pltpu.VMEM((NCHUNK, K), jnp.int32),
          pltpu.VMEM((NBUF * K, D), jnp.float32),
          pltpu.VMEM_SHARED((NP, D), jnp.float32),
          pltpu.SemaphoreType.DMA,
          pltpu.SemaphoreType.DMA,
      ],
  )
  def agg_kernel(row_hbm, col_hbm, src_hbm, zeros_hbm, out_hbm,
                 row_v, col_v, buf, acc, gsem, ssem):
    c = lax.axis_index("c")
    s = lax.axis_index("s")
    t = s * NC + c
    base = s * RPT

    @pl.when(c == 0)
    def _():
      pltpu.sync_copy(src_hbm.at[pl.ds(base, RPT)], acc.at[pl.ds(base, RPT)])

    @pl.when(c != 0)
    def _():
      pltpu.sync_copy(zeros_hbm.at[pl.ds(base, RPT)], acc.at[pl.ds(base, RPT)])

    pltpu.sync_copy(row_hbm.at[t], row_v)
    pltpu.sync_copy(col_hbm.at[t], col_v)
    plsc.subcore_barrier()

    def slot(j):
      return (j % NBUF) * K

    def gissue(j):
      pltpu.async_copy(src_hbm.at[row_v.at[j]],
                       buf.at[pl.ds(slot(j), K)], gsem)

    def gwait(j):
      pltpu.make_async_copy(src_hbm.at[row_v.at[j]],
                            buf.at[pl.ds(slot(j), K)], gsem).wait()

    def sissue(j):
      pltpu.async_copy(buf.at[pl.ds(slot(j), K)],
                       acc.at[col_v.at[j]], ssem, add=True)

    def swait(j):
      pltpu.make_async_copy(buf.at[pl.ds(slot(j), K)],
                            acc.at[col_v.at[j]], ssem).wait()

    for j in range(GD):
      gissue(j)

    def body(j, carry):
      gwait(j)
      sissue(j)

      @pl.when(j + GD < NCHUNK)
      def _():
        @pl.when(j >= SD)
        def _():
          swait(j - SD)

        gissue(j + GD)

      return carry

    lax.fori_loop(0, NCHUNK, body, 0)
    for j in range(max(0, NCHUNK - NBUF), NCHUNK):
      swait(j)
    plsc.subcore_barrier()
    pltpu.sync_copy(acc.at[pl.ds(base, RPT)],
                    out_hbm.at[c].at[pl.ds(base, RPT)])

  return agg_kernel


_deg_kernel = _make_deg_kernel()
_agg128 = _make_agg_kernel(128)
_agg40 = _make_agg_kernel(40)


_RBLK = 640
_NBLK = NP // _RBLK


def _prescale_body(deg_ref, x_ref, xs_ref, dinv_ref):
  deg = deg_ref[:, 0:1] + deg_ref[:, 1:2] + 1.0
  dinv = lax.rsqrt(deg)
  dinv_ref[...] = dinv
  xs_ref[...] = x_ref[...] * dinv


def _tc_prescale(deg_t, x_pad):
  return pl.pallas_call(
      _prescale_body,
      grid=(_NBLK,),
      in_specs=[
          pl.BlockSpec((_RBLK, NC), lambda i: (i, 0)),
          pl.BlockSpec((_RBLK, 128), lambda i: (i, 0)),
      ],
      out_specs=[
          pl.BlockSpec((_RBLK, 128), lambda i: (i, 0)),
          pl.BlockSpec((_RBLK, 1), lambda i: (i, 0)),
      ],
      out_shape=[
          jax.ShapeDtypeStruct((NP, 128), jnp.float32),
          jax.ShapeDtypeStruct((NP, 1), jnp.float32),
      ],
  )(deg_t, x_pad)


def _mm_body(p_ref, dinv_ref, w1_ref, b1_ref, w2_ref, ys_ref):
  dinv = dinv_ref[...]
  a = (p_ref[0] + p_ref[1]) * dinv
  h = jnp.dot(a.astype(jnp.bfloat16), w1_ref[...].astype(jnp.bfloat16),
              preferred_element_type=jnp.float32)
  h = jnp.maximum(h + b1_ref[...], 0.0)
  y = jnp.dot(h.astype(jnp.bfloat16), w2_ref[...].astype(jnp.bfloat16),
              preferred_element_type=jnp.float32)
  ys_ref[...] = y * dinv


def _tc_mm(p, dinv, w1, b1, w2):
  return pl.pallas_call(
      _mm_body,
      grid=(_NBLK,),
      in_specs=[
          pl.BlockSpec((NC, _RBLK, 128), lambda i: (0, i, 0)),
          pl.BlockSpec((_RBLK, 1), lambda i: (i, 0)),
          pl.BlockSpec((128, 2048), lambda i: (0, 0)),
          pl.BlockSpec((1, 2048), lambda i: (0, 0)),
          pl.BlockSpec((2048, 40), lambda i: (0, 0)),
      ],
      out_specs=pl.BlockSpec((_RBLK, 40), lambda i: (i, 0)),
      out_shape=jax.ShapeDtypeStruct((NP, 40), jnp.float32),
  )(p, dinv, w1, b1, w2)


def _final_body(q_ref, dinv_ref, b2_ref, out_ref):
  out_ref[...] = (q_ref[0] + q_ref[1]) * dinv_ref[...] + b2_ref[...]


def _tc_final(q, dinv, b2p):
  return pl.pallas_call(
      _final_body,
      grid=(_NBLK,),
      in_specs=[
          pl.BlockSpec((NC, _RBLK, 40), lambda i: (0, i, 0)),
          pl.BlockSpec((_RBLK, 1), lambda i: (i, 0)),
          pl.BlockSpec((1, 40), lambda i: (0, 0)),
      ],
      out_specs=pl.BlockSpec((_RBLK, 40), lambda i: (i, 0)),
      out_shape=jax.ShapeDtypeStruct((NP, 40), jnp.float32),
  )(q, dinv, b2p)


def kernel(x, edge_index, W1, b1, W2, b2):
  ei = edge_index.astype(jnp.int32)
  row2 = ei[0].reshape(NW, NCHUNK, K)
  col2 = ei[1].reshape(NW, NCHUNK, K)
  x_pad = jnp.pad(x, ((0, NP - N), (0, 0)))
  b1r = b1.reshape(1, 2048)
  b2r = b2.reshape(1, 40)
  ones_k = jnp.ones((K,), jnp.float32)
  z1 = jnp.zeros((NP,), jnp.float32)
  z128 = jnp.zeros((NP, 128), jnp.float32)
  z40 = jnp.zeros((NP, 40), jnp.float32)

  degp = _deg_kernel(col2, ones_k, z1)
  deg_t = degp.T
  xs, dinv = _tc_prescale(deg_t, x_pad)
  p = _agg128(row2, col2, xs, z128)
  ys = _tc_mm(p, dinv, W1, b1r, W2)
  q = _agg40(row2, col2, ys, z40)
  outp = _tc_final(q, dinv, b2r)
  return outp[:N]


# speedup vs baseline: 2.0901x; 1.1017x over previous
import functools

import jax
import jax.numpy as jnp
from jax import lax
from jax.experimental import pallas as pl
from jax.experimental.pallas import tpu as pltpu
from jax.experimental.pallas import tpu_sc as plsc

N = 10000
NP = 10240
E = 160000
NC = 2
NS = 16
NW = NC * NS
EPT = E // NW
K = 40
NCHUNK = EPT // K
RB = 5
NROUND = NCHUNK // RB
NB = 4
RPT = NP // NS

_MESH = plsc.VectorSubcoreMesh(core_axis_name="c", subcore_axis_name="s")


def _make_deg_kernel():
  DDEEP = 4

  @functools.partial(
      pl.kernel,
      out_type=jax.ShapeDtypeStruct((NC, NP), jnp.float32),
      mesh=_MESH,
      compiler_params=pltpu.CompilerParams(use_tc_tiling_on_sc=False),
      scratch_types=[
          pltpu.VMEM((NCHUNK, K), jnp.int32),
          pltpu.VMEM((K,), jnp.float32),
          pltpu.VMEM_SHARED((NP,), jnp.float32),
          pltpu.SemaphoreType.DMA,
      ],
  )
  def deg_kernel(col_hbm, ones_hbm, zeros_hbm, out_hbm, col_v, ones_v, dacc,
                 sem):
    c = lax.axis_index("c")
    s = lax.axis_index("s")
    t = s * NC + c
    base = s * RPT
    pltpu.sync_copy(zeros_hbm.at[pl.ds(base, RPT)], dacc.at[pl.ds(base, RPT)])
    pltpu.sync_copy(col_hbm.at[t], col_v)
    pltpu.sync_copy(ones_hbm, ones_v)
    plsc.subcore_barrier()

    for j in range(DDEEP):
      pltpu.async_copy(ones_v, dacc.at[col_v.at[j]], sem, add=True)

    def body(j, carry):
      @pl.when(j + DDEEP < NCHUNK)
      def _():
        pltpu.async_copy(ones_v, dacc.at[col_v.at[j + DDEEP]], sem, add=True)

      pltpu.make_async_copy(ones_v, dacc.at[col_v.at[j]], sem).wait()
      return carry

    lax.fori_loop(0, NCHUNK, body, 0)
    plsc.subcore_barrier()
    pltpu.sync_copy(dacc.at[pl.ds(base, RPT)],
                    out_hbm.at[c].at[pl.ds(base, RPT)])

  return deg_kernel


def _make_agg_kernel(D, GD, SD):
  NBUF = GD + SD

  @functools.partial(
      pl.kernel,
      out_type=jax.ShapeDtypeStruct((NC, NP, D), jnp.float32),
      mesh=_MESH,
      compiler_params=pltpu.CompilerParams(use_tc_tiling_on_sc=False),
      scratch_types=[
          pltpu.VMEM((NCHUNK, K), jnp.int32),
          pltpu.VMEM((NCHUNK, K), jnp.int32),
          pltpu.VMEM((NBUF * K, D), jnp.float32),
          pltpu.VMEM_SHARED((NP, D), jnp.float32),
          pltpu.SemaphoreType.DMA,
          pltpu.SemaphoreType.DMA,
      ],
  )
  def agg_kernel(row_hbm, col_hbm, src_hbm, zeros_hbm, out_hbm,
                 row_v, col_v, buf, acc, gsem, ssem):
    c = lax.axis_index("c")
    s = lax.axis_index("s")
    t = s * NC + c
    base = s * RPT

    @pl.when(c == 0)
    def _():
      pltpu.sync_copy(src_hbm.at[pl.ds(base, RPT)], acc.at[pl.ds(base, RPT)])

    @pl.when(c != 0)
    def _():
      pltpu.sync_copy(zeros_hbm.at[pl.ds(base, RPT)], acc.at[pl.ds(base, RPT)])

    pltpu.sync_copy(row_hbm.at[t], row_v)
    pltpu.sync_copy(col_hbm.at[t], col_v)
    plsc.subcore_barrier()

    def slot(j):
      return (j % NBUF) * K

    def gissue(j):
      pltpu.async_copy(src_hbm.at[row_v.at[j]],
                       buf.at[pl.ds(slot(j), K)], gsem)

    def gwait(j):
      pltpu.make_async_copy(src_hbm.at[row_v.at[j]],
                            buf.at[pl.ds(slot(j), K)], gsem).wait()

    def sissue(j):
      pltpu.async_copy(buf.at[pl.ds(slot(j), K)],
                       acc.at[col_v.at[j]], ssem, add=True)

    def swait(j):
      pltpu.make_async_copy(buf.at[pl.ds(slot(j), K)],
                            acc.at[col_v.at[j]], ssem).wait()

    for j in range(GD):
      gissue(j)

    def body(j, carry):
      gwait(j)
      sissue(j)

      @pl.when(j + GD < NCHUNK)
      def _():
        @pl.when(j >= SD)
        def _():
          swait(j - SD)

        gissue(j + GD)

      return carry

    lax.fori_loop(0, NCHUNK, body, 0)
    for j in range(max(0, NCHUNK - NBUF), NCHUNK):
      swait(j)
    plsc.subcore_barrier()
    pltpu.sync_copy(acc.at[pl.ds(base, RPT)],
                    out_hbm.at[c].at[pl.ds(base, RPT)])

  return agg_kernel


_deg_kernel = _make_deg_kernel()
_agg128 = _make_agg_kernel(128, 4, 3)
_agg40 = _make_agg_kernel(40, 5, 5)


_RBLK = 640
_NBLK = NP // _RBLK


def _prescale_body(deg_ref, x_ref, xs_ref, dinv_ref):
  deg = deg_ref[:, 0:1] + deg_ref[:, 1:2] + 1.0
  dinv = lax.rsqrt(deg)
  dinv_ref[...] = dinv
  xs_ref[...] = x_ref[...] * dinv


def _tc_prescale(deg_t, x_pad):
  return pl.pallas_call(
      _prescale_body,
      grid=(_NBLK,),
      in_specs=[
          pl.BlockSpec((_RBLK, NC), lambda i: (i, 0)),
          pl.BlockSpec((_RBLK, 128), lambda i: (i, 0)),
      ],
      out_specs=[
          pl.BlockSpec((_RBLK, 128), lambda i: (i, 0)),
          pl.BlockSpec((_RBLK, 1), lambda i: (i, 0)),
      ],
      out_shape=[
          jax.ShapeDtypeStruct((NP, 128), jnp.float32),
          jax.ShapeDtypeStruct((NP, 1), jnp.float32),
      ],
  )(deg_t, x_pad)


def _mm_body(p_ref, dinv_ref, w1_ref, b1_ref, w2_ref, ys_ref):
  dinv = dinv_ref[...]
  a = (p_ref[0] + p_ref[1]) * dinv
  h = jnp.dot(a.astype(jnp.bfloat16), w1_ref[...].astype(jnp.bfloat16),
              preferred_element_type=jnp.float32)
  h = jnp.maximum(h + b1_ref[...], 0.0)
  y = jnp.dot(h.astype(jnp.bfloat16), w2_ref[...].astype(jnp.bfloat16),
              preferred_element_type=jnp.float32)
  ys_ref[...] = y * dinv


_MBLK = 1280


def _tc_mm(p, dinv, w1, b1, w2):
  return pl.pallas_call(
      _mm_body,
      grid=(NP // _MBLK,),
      in_specs=[
          pl.BlockSpec((NC, _MBLK, 128), lambda i: (0, i, 0)),
          pl.BlockSpec((_MBLK, 1), lambda i: (i, 0)),
          pl.BlockSpec((128, 2048), lambda i: (0, 0)),
          pl.BlockSpec((1, 2048), lambda i: (0, 0)),
          pl.BlockSpec((2048, 40), lambda i: (0, 0)),
      ],
      out_specs=pl.BlockSpec((_MBLK, 40), lambda i: (i, 0)),
      out_shape=jax.ShapeDtypeStruct((NP, 40), jnp.float32),
  )(p, dinv, w1, b1, w2)


def _final_body(q_ref, dinv_ref, b2_ref, out_ref):
  out_ref[...] = (q_ref[0] + q_ref[1]) * dinv_ref[...] + b2_ref[...]


def _tc_final(q, dinv, b2p):
  return pl.pallas_call(
      _final_body,
      grid=(_NBLK,),
      in_specs=[
          pl.BlockSpec((NC, _RBLK, 40), lambda i: (0, i, 0)),
          pl.BlockSpec((_RBLK, 1), lambda i: (i, 0)),
          pl.BlockSpec((1, 40), lambda i: (0, 0)),
      ],
      out_specs=pl.BlockSpec((_RBLK, 40), lambda i: (i, 0)),
      out_shape=jax.ShapeDtypeStruct((NP, 40), jnp.float32),
  )(q, dinv, b2p)


def kernel(x, edge_index, W1, b1, W2, b2):
  ei = edge_index.astype(jnp.int32)
  row2 = ei[0].reshape(NW, NCHUNK, K)
  col2 = ei[1].reshape(NW, NCHUNK, K)
  x_pad = jnp.pad(x, ((0, NP - N), (0, 0)))
  b1r = b1.reshape(1, 2048)
  b2r = b2.reshape(1, 40)
  ones_k = jnp.ones((K,), jnp.float32)
  z1 = jnp.zeros((NP,), jnp.float32)
  z128 = jnp.zeros((NP, 128), jnp.float32)
  z40 = jnp.zeros((NP, 40), jnp.float32)

  degp = _deg_kernel(col2, ones_k, z1)
  deg_t = degp.T
  xs, dinv = _tc_prescale(deg_t, x_pad)
  p = _agg128(row2, col2, xs, z128)
  ys = _tc_mm(p, dinv, W1, b1r, W2)
  q = _agg40(row2, col2, ys, z40)
  outp = _tc_final(q, dinv, b2r)
  return outp[:N]
